# baseline (speedup 1.0000x reference)
import jax
import jax.numpy as jnp
from jax import lax
from jax.experimental import pallas as pl
from jax.experimental.pallas import tpu as pltpu
from jax.experimental.pallas import tpu_sc as plsc

N = 10000
E = 320000
D = 128
B = 128

TPS = 16
NW = 2 * TPS
C = 128
K = 80
EPT = K * C
E_PAD = NW * EPT
NPAD = 10240
RPT = NPAD // TPS
DE = 4

R = 512
NB = NPAD // R


def _sc_body(x_hbm, idx_hbm, ea_hbm, zt_hbm,
             t0_hbm, t1_hbm, a0_hbm, a1_hbm,
             acc_sh, rows2, eb2, idx2, gsem, isem):
    c = lax.axis_index("c")
    s = lax.axis_index("s")
    wid = c * TPS + s
    row0 = s * RPT

    pltpu.sync_copy(zt_hbm.at[pl.ds(row0, RPT)], acc_sh.at[pl.ds(row0, RPT)])
    plsc.subcore_barrier()

    pltpu.sync_copy(idx_hbm.at[wid, 0], idx2.at[0])
    pltpu.async_copy(x_hbm.at[idx2.at[0, 0]], rows2.at[0], gsem)

    def tpair(i, carry):
        for b in range(2):
            j = 2 * i + b
            nb = 1 - b
            last = (j + 1 >= K) if b else False

            def stage():
                pltpu.async_copy(idx_hbm.at[wid, j + 1], idx2.at[nb], isem)
                pltpu.make_async_copy(
                    x_hbm.at[idx2.at[b, 0]], rows2.at[b], gsem).wait()
                pltpu.make_async_copy(
                    idx_hbm.at[wid, 0], idx2.at[nb], isem).wait()
                pltpu.async_copy(x_hbm.at[idx2.at[nb, 0]], rows2.at[nb], gsem)

            if b == 0:
                stage()
            else:
                @pl.when(i < K // 2 - 1)
                def _():
                    stage()

                @pl.when(i >= K // 2 - 1)
                def _():
                    pltpu.make_async_copy(
                        x_hbm.at[idx2.at[b, 0]], rows2.at[b], gsem).wait()

            pltpu.sync_copy(rows2.at[b], acc_sh.at[idx2.at[b, 1]], add=True)
        return carry

    lax.fori_loop(0, K // 2, tpair, 0)
    plsc.subcore_barrier()

    @pl.when(c == 0)
    def _():
        pltpu.sync_copy(acc_sh.at[pl.ds(row0, RPT)], t0_hbm.at[pl.ds(row0, RPT)])

    @pl.when(c == 1)
    def _():
        pltpu.sync_copy(acc_sh.at[pl.ds(row0, RPT)], t1_hbm.at[pl.ds(row0, RPT)])

    plsc.subcore_barrier()

    pltpu.sync_copy(zt_hbm.at[pl.ds(row0, RPT)], acc_sh.at[pl.ds(row0, RPT)])
    pltpu.sync_copy(zt_hbm.at[pl.ds(0, C)], rows2.at[0])
    pltpu.sync_copy(zt_hbm.at[pl.ds(0, C)], rows2.at[1])
    plsc.subcore_barrier()

    pltpu.sync_copy(idx_hbm.at[wid, 0], idx2.at[0])
    pltpu.async_copy(ea_hbm.at[wid, 0], eb2.at[0], gsem)

    def apair(i, carry):
        for b in range(2):
            j = 2 * i + b
            nb = 1 - b

            def stage():
                pltpu.async_copy(idx_hbm.at[wid, j + 1], idx2.at[nb], isem)
                pltpu.async_copy(ea_hbm.at[wid, j + 1], eb2.at[nb], gsem)

            if b == 0:
                stage()
            else:
                @pl.when(i < K // 2 - 1)
                def _():
                    stage()

            pltpu.make_async_copy(
                ea_hbm.at[wid, 0], eb2.at[b], gsem).wait()
            for j8 in range(C // 8):
                for m in range(8):
                    rows2[b, 8 * j8 + m, pl.ds(0, 16)] = (
                        eb2[b, j8, pl.ds(16 * m, 16)])
            if b == 0:
                pltpu.make_async_copy(
                    idx_hbm.at[wid, 0], idx2.at[nb], isem).wait()
            else:
                @pl.when(i < K // 2 - 1)
                def _():
                    pltpu.make_async_copy(
                        idx_hbm.at[wid, 0], idx2.at[nb], isem).wait()
            pltpu.sync_copy(rows2.at[b], acc_sh.at[idx2.at[b, 1]], add=True)
        return carry

    lax.fori_loop(0, K // 2, apair, 0)
    plsc.subcore_barrier()

    @pl.when(c == 0)
    def _():
        pltpu.sync_copy(acc_sh.at[pl.ds(row0, RPT)], a0_hbm.at[pl.ds(row0, RPT)])

    @pl.when(c == 1)
    def _():
        pltpu.sync_copy(acc_sh.at[pl.ds(row0, RPT)], a1_hbm.at[pl.ds(row0, RPT)])


def _sc_scatter():
  return pl.kernel(
    _sc_body,
    out_type=[
        jax.ShapeDtypeStruct((NPAD, D), jnp.float32),
        jax.ShapeDtypeStruct((NPAD, D), jnp.float32),
        jax.ShapeDtypeStruct((NPAD, D), jnp.float32),
        jax.ShapeDtypeStruct((NPAD, D), jnp.float32),
    ],
    mesh=plsc.VectorSubcoreMesh(core_axis_name="c", subcore_axis_name="s",
                                num_cores=2, num_subcores=TPS),
    scratch_types=[
        pltpu.VMEM_SHARED((NPAD, D), jnp.float32),
        pltpu.VMEM((2, C, D), jnp.float32),
        pltpu.VMEM((2, 16, 128), jnp.float32),
        pltpu.VMEM((2, 2, C), jnp.int32),
        pltpu.SemaphoreType.DMA,
        pltpu.SemaphoreType.DMA,
    ],
  )


def _tc_body(x_r, t0_r, t1_r, a0_r, a1_r, brow_r, bcol_r,
             wei_r, wni_r, weg_r, wng_r, w1_r, b1_r, w2_r, b2_r,
             z_r, seg, cnt, pooled):
    g = pl.program_id(0)
    f32 = jnp.float32
    NEG = jnp.float32(-3.0e38)

    @pl.when(g == 0)
    def _():
        seg[...] = jnp.full((1, B), NEG, f32)
        cnt[...] = jnp.zeros((B, 1), f32)
        pooled[...] = jnp.zeros((B, D), f32)

    @pl.when(g < 2 * NB)
    def _():
        xs = x_r[...] + t0_r[...] + t1_r[...]
        a = a0_r[...] + a1_r[...]
        ni_full = (xs + a @ wei_r[...]) @ wni_r[...]
        ni = jnp.sum(ni_full, axis=1, keepdims=True)
        bcol = bcol_r[...]
        m2 = jnp.broadcast_to(bcol, (R, B)) == lax.broadcasted_iota(
            jnp.int32, (R, B), 1)
        m2f = m2.astype(f32)

        @pl.when(g < NB)
        def _():
            cand = jnp.where(m2, jnp.broadcast_to(ni, (R, B)), NEG)
            seg[...] = jnp.maximum(seg[...], jnp.max(cand, axis=0, keepdims=True))
            brow = brow_r[...]
            m1 = jnp.broadcast_to(brow, (B, R)) == lax.broadcasted_iota(
                jnp.int32, (B, R), 0)
            cnt[...] += jnp.sum(m1.astype(f32), axis=1, keepdims=True)

        @pl.when(g >= NB)
        def _():
            segrow = jnp.broadcast_to(seg[...], (R, B))
            outc = jnp.sum(m2f * segrow, axis=1, keepdims=True)
            outc = outc + (1.0 - jnp.sum(m2f, axis=1, keepdims=True))
            imp = ni / (outc * 10.0) + 0.9
            h = jnp.maximum((xs + a @ weg_r[...]) @ wng_r[...], 0.0)
            hi = h * imp
            brow = brow_r[...]
            m1 = jnp.broadcast_to(brow, (B, R)) == lax.broadcasted_iota(
                jnp.int32, (B, R), 0)
            pooled[...] += jnp.dot(m1.astype(f32), hi,
                                   preferred_element_type=f32)

    @pl.when(g == 2 * NB)
    def _():
        pm = pooled[...] / jnp.maximum(cnt[...], 1.0)
        z1 = jnp.maximum(pm @ w1_r[...] + b1_r[...], 0.0)
        z_r[...] = z1 @ w2_r[...] + b2_r[...]


def _tc_call(xp, t0, t1, a0, a1, brow, bcol, wei, wni, weg, wng, w1, b1r,
             w2, b2r):
    blk = lambda g: (lax.rem(g, NB), 0)
    full = lambda g: (0, 0)
    return pl.pallas_call(
        _tc_body,
        grid=(2 * NB + 1,),
        in_specs=[
            pl.BlockSpec((R, D), blk),
            pl.BlockSpec((R, D), blk),
            pl.BlockSpec((R, D), blk),
            pl.BlockSpec((R, D), blk),
            pl.BlockSpec((R, D), blk),
            pl.BlockSpec((1, R), lambda g: (0, lax.rem(g, NB))),
            pl.BlockSpec((R, 1), blk),
            pl.BlockSpec((D, D), full),
            pl.BlockSpec((D, D), full),
            pl.BlockSpec((D, D), full),
            pl.BlockSpec((D, D), full),
            pl.BlockSpec((D, D), full),
            pl.BlockSpec((1, D), full),
            pl.BlockSpec((D, D), full),
            pl.BlockSpec((1, D), full),
        ],
        out_specs=pl.BlockSpec((B, D), full),
        out_shape=jax.ShapeDtypeStruct((B, D), jnp.float32),
        scratch_shapes=[
            pltpu.VMEM((1, B), jnp.float32),
            pltpu.VMEM((B, 1), jnp.float32),
            pltpu.VMEM((B, D), jnp.float32),
        ],
        compiler_params=pltpu.CompilerParams(
            dimension_semantics=("arbitrary",)),
    )(xp, t0, t1, a0, a1, brow, bcol, wei, wni, weg, wng, w1, b1r, w2, b2r)


def kernel(x, edge_index, edge_attr, batch, We_imp, Wn_imp, We_g, Wn_g,
           W1, b1, W2, b2):
    pad_e = E_PAD - E
    src = jnp.concatenate([edge_index[0], jnp.zeros((pad_e,), jnp.int32)])
    dst = jnp.concatenate([edge_index[1],
                           jnp.full((pad_e,), N, jnp.int32)])
    idxp = jnp.stack([src.reshape(NW, K, C), dst.reshape(NW, K, C)], axis=2)
    eap = jnp.pad(edge_attr, ((0, pad_e), (0, 16 - DE)))
    eap = eap.reshape(NW, K, 16, 128)
    zt = jnp.zeros((NPAD, D), jnp.float32)

    t0, t1, a0, a1 = _sc_scatter()(x, idxp, eap, zt)

    xp = jnp.pad(x, ((0, NPAD - N), (0, 0)))
    bpad = jnp.pad(batch, (0, NPAD - N), constant_values=B)
    brow = bpad.reshape(1, NPAD)
    bcol = bpad.reshape(NPAD, 1)
    wei = jnp.pad(We_imp, ((0, D - We_imp.shape[0]), (0, 0)))
    weg = jnp.pad(We_g, ((0, D - We_g.shape[0]), (0, 0)))
    wni = jnp.pad(Wn_imp, ((0, 0), (0, D - Wn_imp.shape[1])))
    b1r = b1.reshape(1, D)
    b2r = b2.reshape(1, D)

    return _tc_call(xp, t0, t1, a0, a1, brow, bcol, wei, wni, weg, Wn_g,
                    W1, b1r, W2, b2r)


# speedup vs baseline: 1.0568x; 1.0568x over previous
import jax
import jax.numpy as jnp
from jax import lax
from jax.experimental import pallas as pl
from jax.experimental.pallas import tpu as pltpu
from jax.experimental.pallas import tpu_sc as plsc

N = 10000
E = 320000
D = 128
B = 128

TPS = 16
C = 128
K = 160
EPT = K * C
E_PAD = TPS * EPT
NPAD = 10240
RPT = NPAD // TPS
DE = 4

R = 512
NB = NPAD // R


def _sc_body(x_hbm, idx_hbm, ea_hbm, zt_hbm,
             t_hbm, a_hbm,
             acc_sh, rows2, eb2, idx2, gsem, isem):
    c = lax.axis_index("c")
    s = lax.axis_index("s")
    row0 = s * RPT

    pltpu.sync_copy(zt_hbm.at[pl.ds(row0, RPT)], acc_sh.at[pl.ds(row0, RPT)])

    @pl.when(c == 1)
    def _():
        pltpu.sync_copy(zt_hbm.at[pl.ds(0, C)], rows2.at[0])
        pltpu.sync_copy(zt_hbm.at[pl.ds(0, C)], rows2.at[1])

    plsc.subcore_barrier()

    @pl.when(c == 0)
    def _():
        pltpu.sync_copy(idx_hbm.at[s, 0], idx2.at[0])
        pltpu.async_copy(x_hbm.at[idx2.at[0, 0]], rows2.at[0], gsem)

        def tpair(i, carry):
            for b in range(2):
                j = 2 * i + b
                nb = 1 - b

                def stage():
                    pltpu.async_copy(idx_hbm.at[s, j + 1], idx2.at[nb], isem)
                    pltpu.make_async_copy(
                        x_hbm.at[idx2.at[b, 0]], rows2.at[b], gsem).wait()
                    pltpu.make_async_copy(
                        idx_hbm.at[s, 0], idx2.at[nb], isem).wait()
                    pltpu.async_copy(
                        x_hbm.at[idx2.at[nb, 0]], rows2.at[nb], gsem)

                if b == 0:
                    stage()
                else:
                    @pl.when(i < K // 2 - 1)
                    def _():
                        stage()

                    @pl.when(i >= K // 2 - 1)
                    def _():
                        pltpu.make_async_copy(
                            x_hbm.at[idx2.at[b, 0]], rows2.at[b], gsem).wait()

                pltpu.sync_copy(rows2.at[b], acc_sh.at[idx2.at[b, 1]],
                                add=True)
            return carry

        lax.fori_loop(0, K // 2, tpair, 0)

    @pl.when(c == 1)
    def _():
        pltpu.sync_copy(idx_hbm.at[s, 0], idx2.at[0])
        pltpu.async_copy(ea_hbm.at[s, 0], eb2.at[0], gsem)

        def apair(i, carry):
            for b in range(2):
                j = 2 * i + b
                nb = 1 - b

                def stage():
                    pltpu.async_copy(idx_hbm.at[s, j + 1], idx2.at[nb], isem)
                    pltpu.async_copy(ea_hbm.at[s, j + 1], eb2.at[nb], gsem)

                if b == 0:
                    stage()
                else:
                    @pl.when(i < K // 2 - 1)
                    def _():
                        stage()

                pltpu.make_async_copy(
                    ea_hbm.at[s, 0], eb2.at[b], gsem).wait()
                for j8 in range(C // 8):
                    for m in range(8):
                        rows2[b, 8 * j8 + m, pl.ds(0, 16)] = (
                            eb2[b, j8, pl.ds(16 * m, 16)])
                if b == 0:
                    pltpu.make_async_copy(
                        idx_hbm.at[s, 0], idx2.at[nb], isem).wait()
                else:
                    @pl.when(i < K // 2 - 1)
                    def _():
                        pltpu.make_async_copy(
                            idx_hbm.at[s, 0], idx2.at[nb], isem).wait()
                pltpu.sync_copy(rows2.at[b], acc_sh.at[idx2.at[b, 1]],
                                add=True)
            return carry

        lax.fori_loop(0, K // 2, apair, 0)

    plsc.subcore_barrier()

    @pl.when(c == 0)
    def _():
        pltpu.sync_copy(acc_sh.at[pl.ds(row0, RPT)], t_hbm.at[pl.ds(row0, RPT)])

    @pl.when(c == 1)
    def _():
        pltpu.sync_copy(acc_sh.at[pl.ds(row0, RPT)], a_hbm.at[pl.ds(row0, RPT)])


def _sc_scatter():
  return pl.kernel(
    _sc_body,
    out_type=[
        jax.ShapeDtypeStruct((NPAD, D), jnp.float32),
        jax.ShapeDtypeStruct((NPAD, D), jnp.float32),
    ],
    mesh=plsc.VectorSubcoreMesh(core_axis_name="c", subcore_axis_name="s",
                                num_cores=2, num_subcores=TPS),
    scratch_types=[
        pltpu.VMEM_SHARED((NPAD, D), jnp.float32),
        pltpu.VMEM((2, C, D), jnp.float32),
        pltpu.VMEM((2, 16, 128), jnp.float32),
        pltpu.VMEM((2, 2, C), jnp.int32),
        pltpu.SemaphoreType.DMA,
        pltpu.SemaphoreType.DMA,
    ],
  )


def _tc_body(x_r, t_r, a_r, brow_r, bcol_r,
             wei_r, wni_r, weg_r, wng_r, w1_r, b1_r, w2_r, b2_r,
             z_r, seg, cnt, pooled):
    g = pl.program_id(0)
    f32 = jnp.float32
    NEG = jnp.float32(-3.0e38)

    @pl.when(g == 0)
    def _():
        seg[...] = jnp.full((1, B), NEG, f32)
        cnt[...] = jnp.zeros((B, 1), f32)
        pooled[...] = jnp.zeros((B, D), f32)

    @pl.when(g < 2 * NB)
    def _():
        xs = x_r[...] + t_r[...]
        a = a_r[...]
        ni_full = (xs + a @ wei_r[...]) @ wni_r[...]
        ni = jnp.sum(ni_full, axis=1, keepdims=True)
        bcol = bcol_r[...]
        m2 = jnp.broadcast_to(bcol, (R, B)) == lax.broadcasted_iota(
            jnp.int32, (R, B), 1)
        m2f = m2.astype(f32)

        @pl.when(g < NB)
        def _():
            cand = jnp.where(m2, jnp.broadcast_to(ni, (R, B)), NEG)
            seg[...] = jnp.maximum(seg[...], jnp.max(cand, axis=0, keepdims=True))
            brow = brow_r[...]
            m1 = jnp.broadcast_to(brow, (B, R)) == lax.broadcasted_iota(
                jnp.int32, (B, R), 0)
            cnt[...] += jnp.sum(m1.astype(f32), axis=1, keepdims=True)

        @pl.when(g >= NB)
        def _():
            segrow = jnp.broadcast_to(seg[...], (R, B))
            outc = jnp.sum(m2f * segrow, axis=1, keepdims=True)
            outc = outc + (1.0 - jnp.sum(m2f, axis=1, keepdims=True))
            imp = ni / (outc * 10.0) + 0.9
            h = jnp.maximum((xs + a @ weg_r[...]) @ wng_r[...], 0.0)
            hi = h * imp
            brow = brow_r[...]
            m1 = jnp.broadcast_to(brow, (B, R)) == lax.broadcasted_iota(
                jnp.int32, (B, R), 0)
            pooled[...] += jnp.dot(m1.astype(f32), hi,
                                   preferred_element_type=f32)

    @pl.when(g == 2 * NB)
    def _():
        pm = pooled[...] / jnp.maximum(cnt[...], 1.0)
        z1 = jnp.maximum(pm @ w1_r[...] + b1_r[...], 0.0)
        z_r[...] = z1 @ w2_r[...] + b2_r[...]


def _tc_call(xp, t, a, brow, bcol, wei, wni, weg, wng, w1, b1r, w2, b2r):
    blk = lambda g: (lax.rem(g, NB), 0)
    full = lambda g: (0, 0)
    return pl.pallas_call(
        _tc_body,
        grid=(2 * NB + 1,),
        in_specs=[
            pl.BlockSpec((R, D), blk),
            pl.BlockSpec((R, D), blk),
            pl.BlockSpec((R, D), blk),
            pl.BlockSpec((1, R), lambda g: (0, lax.rem(g, NB))),
            pl.BlockSpec((R, 1), blk),
            pl.BlockSpec((D, D), full),
            pl.BlockSpec((D, D), full),
            pl.BlockSpec((D, D), full),
            pl.BlockSpec((D, D), full),
            pl.BlockSpec((D, D), full),
            pl.BlockSpec((1, D), full),
            pl.BlockSpec((D, D), full),
            pl.BlockSpec((1, D), full),
        ],
        out_specs=pl.BlockSpec((B, D), full),
        out_shape=jax.ShapeDtypeStruct((B, D), jnp.float32),
        scratch_shapes=[
            pltpu.VMEM((1, B), jnp.float32),
            pltpu.VMEM((B, 1), jnp.float32),
            pltpu.VMEM((B, D), jnp.float32),
        ],
        compiler_params=pltpu.CompilerParams(
            dimension_semantics=("arbitrary",)),
    )(xp, t, a, brow, bcol, wei, wni, weg, wng, w1, b1r, w2, b2r)


def kernel(x, edge_index, edge_attr, batch, We_imp, Wn_imp, We_g, Wn_g,
           W1, b1, W2, b2):
    pad_e = E_PAD - E
    src = jnp.concatenate([edge_index[0], jnp.zeros((pad_e,), jnp.int32)])
    dst = jnp.concatenate([edge_index[1],
                           jnp.full((pad_e,), N, jnp.int32)])
    idxp = jnp.stack([src.reshape(TPS, K, C), dst.reshape(TPS, K, C)], axis=2)
    eap = jnp.pad(edge_attr, ((0, pad_e), (0, 16 - DE)))
    eap = eap.reshape(TPS, K, 16, 128)
    zt = jnp.zeros((NPAD, D), jnp.float32)

    t, a = _sc_scatter()(x, idxp, eap, zt)

    xp = jnp.pad(x, ((0, NPAD - N), (0, 0)))
    bpad = jnp.pad(batch, (0, NPAD - N), constant_values=B)
    brow = bpad.reshape(1, NPAD)
    bcol = bpad.reshape(NPAD, 1)
    wei = jnp.pad(We_imp, ((0, D - We_imp.shape[0]), (0, 0)))
    weg = jnp.pad(We_g, ((0, D - We_g.shape[0]), (0, 0)))
    wni = jnp.pad(Wn_imp, ((0, 0), (0, D - Wn_imp.shape[1])))
    b1r = b1.reshape(1, D)
    b2r = b2.reshape(1, D)

    return _tc_call(xp, t, a, brow, bcol, wei, wni, weg, Wn_g,
                    W1, b1r, W2, b2r)


# speedup vs baseline: 1.0751x; 1.0173x over previous
import jax
import jax.numpy as jnp
from jax import lax
from jax.experimental import pallas as pl
from jax.experimental.pallas import tpu as pltpu
from jax.experimental.pallas import tpu_sc as plsc

N = 10000
E = 320000
D = 128
B = 128

TPS = 16
C = 128
K = 160
EPT = K * C
E_PAD = TPS * EPT
NPAD = 10240
RPT = NPAD // TPS
DE = 4

R = 1024
NB = NPAD // R


def _sc_body(x_hbm, idx_hbm, ea_hbm, zt_hbm,
             t_hbm, a_hbm,
             acc_sh, rows2, eb2, idx2, gsem, isem):
    c = lax.axis_index("c")
    s = lax.axis_index("s")
    row0 = s * RPT

    pltpu.sync_copy(zt_hbm.at[pl.ds(row0, RPT)], acc_sh.at[pl.ds(row0, RPT)])

    @pl.when(c == 1)
    def _():
        pltpu.sync_copy(zt_hbm.at[pl.ds(0, C)], rows2.at[0])
        pltpu.sync_copy(zt_hbm.at[pl.ds(0, C)], rows2.at[1])

    plsc.subcore_barrier()

    @pl.when(c == 0)
    def _():
        pltpu.sync_copy(idx_hbm.at[s, 0], idx2.at[0])
        pltpu.async_copy(x_hbm.at[idx2.at[0, 0]], rows2.at[0], gsem)

        def tpair(i, carry):
            for b in range(2):
                j = 2 * i + b
                nb = 1 - b

                def stage():
                    pltpu.async_copy(idx_hbm.at[s, j + 1], idx2.at[nb], isem)
                    pltpu.make_async_copy(
                        x_hbm.at[idx2.at[b, 0]], rows2.at[b], gsem).wait()
                    pltpu.make_async_copy(
                        idx_hbm.at[s, 0], idx2.at[nb], isem).wait()
                    pltpu.async_copy(
                        x_hbm.at[idx2.at[nb, 0]], rows2.at[nb], gsem)

                if b == 0:
                    stage()
                else:
                    @pl.when(i < K // 2 - 1)
                    def _():
                        stage()

                    @pl.when(i >= K // 2 - 1)
                    def _():
                        pltpu.make_async_copy(
                            x_hbm.at[idx2.at[b, 0]], rows2.at[b], gsem).wait()

                pltpu.sync_copy(rows2.at[b], acc_sh.at[idx2.at[b, 1]],
                                add=True)
            return carry

        lax.fori_loop(0, K // 2, tpair, 0)

    @pl.when(c == 1)
    def _():
        pltpu.sync_copy(idx_hbm.at[s, 0], idx2.at[0])
        pltpu.async_copy(ea_hbm.at[s, 0], eb2.at[0], gsem)

        def apair(i, carry):
            for b in range(2):
                j = 2 * i + b
                nb = 1 - b

                def stage():
                    pltpu.async_copy(idx_hbm.at[s, j + 1], idx2.at[nb], isem)
                    pltpu.async_copy(ea_hbm.at[s, j + 1], eb2.at[nb], gsem)

                if b == 0:
                    stage()
                else:
                    @pl.when(i < K // 2 - 1)
                    def _():
                        stage()

                pltpu.make_async_copy(
                    ea_hbm.at[s, 0], eb2.at[b], gsem).wait()
                for j8 in range(C // 8):
                    for m in range(8):
                        rows2[b, 8 * j8 + m, pl.ds(0, 16)] = (
                            eb2[b, j8, pl.ds(16 * m, 16)])
                if b == 0:
                    pltpu.make_async_copy(
                        idx_hbm.at[s, 0], idx2.at[nb], isem).wait()
                else:
                    @pl.when(i < K // 2 - 1)
                    def _():
                        pltpu.make_async_copy(
                            idx_hbm.at[s, 0], idx2.at[nb], isem).wait()
                pltpu.sync_copy(rows2.at[b], acc_sh.at[idx2.at[b, 1]],
                                add=True)
            return carry

        lax.fori_loop(0, K // 2, apair, 0)

    plsc.subcore_barrier()

    @pl.when(c == 0)
    def _():
        pltpu.sync_copy(acc_sh.at[pl.ds(row0, RPT)], t_hbm.at[pl.ds(row0, RPT)])

    @pl.when(c == 1)
    def _():
        pltpu.sync_copy(acc_sh.at[pl.ds(row0, RPT)], a_hbm.at[pl.ds(row0, RPT)])


def _sc_scatter():
  return pl.kernel(
    _sc_body,
    out_type=[
        jax.ShapeDtypeStruct((NPAD, D), jnp.float32),
        jax.ShapeDtypeStruct((NPAD, D), jnp.float32),
    ],
    mesh=plsc.VectorSubcoreMesh(core_axis_name="c", subcore_axis_name="s",
                                num_cores=2, num_subcores=TPS),
    scratch_types=[
        pltpu.VMEM_SHARED((NPAD, D), jnp.float32),
        pltpu.VMEM((2, C, D), jnp.float32),
        pltpu.VMEM((2, 16, 128), jnp.float32),
        pltpu.VMEM((2, 2, C), jnp.int32),
        pltpu.SemaphoreType.DMA,
        pltpu.SemaphoreType.DMA,
    ],
  )


def _tc_body(x_r, t_r, a_r, brow_r, bcol_r,
             wei_r, wni_r, weg_r, wng_r, w1_r, b1_r, w2_r, b2_r,
             z_r, seg, cnt, pooled):
    g = pl.program_id(0)
    f32 = jnp.float32
    NEG = jnp.float32(-3.0e38)

    @pl.when(g == 0)
    def _():
        seg[...] = jnp.full((1, B), NEG, f32)
        cnt[...] = jnp.zeros((B, 1), f32)
        pooled[...] = jnp.zeros((B, D), f32)

    @pl.when(g < 2 * NB)
    def _():
        xs = x_r[...] + t_r[...]
        a = a_r[...]
        ni_full = (xs + a @ wei_r[...]) @ wni_r[...]
        ni = jnp.sum(ni_full, axis=1, keepdims=True)
        bcol = bcol_r[...]
        m2 = jnp.broadcast_to(bcol, (R, B)) == lax.broadcasted_iota(
            jnp.int32, (R, B), 1)
        m2f = m2.astype(f32)

        @pl.when(g < NB)
        def _():
            cand = jnp.where(m2, jnp.broadcast_to(ni, (R, B)), NEG)
            seg[...] = jnp.maximum(seg[...], jnp.max(cand, axis=0, keepdims=True))
            brow = brow_r[...]
            m1 = jnp.broadcast_to(brow, (B, R)) == lax.broadcasted_iota(
                jnp.int32, (B, R), 0)
            cnt[...] += jnp.sum(m1.astype(f32), axis=1, keepdims=True)

        @pl.when(g >= NB)
        def _():
            segrow = jnp.broadcast_to(seg[...], (R, B))
            outc = jnp.sum(m2f * segrow, axis=1, keepdims=True)
            outc = outc + (1.0 - jnp.sum(m2f, axis=1, keepdims=True))
            imp = ni / (outc * 10.0) + 0.9
            h = jnp.maximum((xs + a @ weg_r[...]) @ wng_r[...], 0.0)
            hi = h * imp
            brow = brow_r[...]
            m1 = jnp.broadcast_to(brow, (B, R)) == lax.broadcasted_iota(
                jnp.int32, (B, R), 0)
            pooled[...] += jnp.dot(m1.astype(f32), hi,
                                   preferred_element_type=f32)

    @pl.when(g == 2 * NB)
    def _():
        pm = pooled[...] / jnp.maximum(cnt[...], 1.0)
        z1 = jnp.maximum(pm @ w1_r[...] + b1_r[...], 0.0)
        z_r[...] = z1 @ w2_r[...] + b2_r[...]


def _tc_call(xp, t, a, brow, bcol, wei, wni, weg, wng, w1, b1r, w2, b2r):
    blk = lambda g: (lax.rem(g, NB), 0)
    full = lambda g: (0, 0)
    return pl.pallas_call(
        _tc_body,
        grid=(2 * NB + 1,),
        in_specs=[
            pl.BlockSpec((R, D), blk),
            pl.BlockSpec((R, D), blk),
            pl.BlockSpec((R, D), blk),
            pl.BlockSpec((1, R), lambda g: (0, lax.rem(g, NB))),
            pl.BlockSpec((R, 1), blk),
            pl.BlockSpec((D, D), full),
            pl.BlockSpec((D, D), full),
            pl.BlockSpec((D, D), full),
            pl.BlockSpec((D, D), full),
            pl.BlockSpec((D, D), full),
            pl.BlockSpec((1, D), full),
            pl.BlockSpec((D, D), full),
            pl.BlockSpec((1, D), full),
        ],
        out_specs=pl.BlockSpec((B, D), full),
        out_shape=jax.ShapeDtypeStruct((B, D), jnp.float32),
        scratch_shapes=[
            pltpu.VMEM((1, B), jnp.float32),
            pltpu.VMEM((B, 1), jnp.float32),
            pltpu.VMEM((B, D), jnp.float32),
        ],
        compiler_params=pltpu.CompilerParams(
            dimension_semantics=("arbitrary",)),
    )(xp, t, a, brow, bcol, wei, wni, weg, wng, w1, b1r, w2, b2r)


def kernel(x, edge_index, edge_attr, batch, We_imp, Wn_imp, We_g, Wn_g,
           W1, b1, W2, b2):
    pad_e = E_PAD - E
    src = jnp.concatenate([edge_index[0], jnp.zeros((pad_e,), jnp.int32)])
    dst = jnp.concatenate([edge_index[1],
                           jnp.full((pad_e,), N, jnp.int32)])
    idxp = jnp.stack([src.reshape(TPS, K, C), dst.reshape(TPS, K, C)], axis=2)
    eap = jnp.pad(edge_attr, ((0, pad_e), (0, 16 - DE)))
    eap = eap.reshape(TPS, K, 16, 128)
    zt = jnp.zeros((NPAD, D), jnp.float32)

    t, a = _sc_scatter()(x, idxp, eap, zt)

    xp = jnp.pad(x, ((0, NPAD - N), (0, 0)))
    bpad = jnp.pad(batch, (0, NPAD - N), constant_values=B)
    brow = bpad.reshape(1, NPAD)
    bcol = bpad.reshape(NPAD, 1)
    wei = jnp.pad(We_imp, ((0, D - We_imp.shape[0]), (0, 0)))
    weg = jnp.pad(We_g, ((0, D - We_g.shape[0]), (0, 0)))
    wni = jnp.pad(Wn_imp, ((0, 0), (0, D - Wn_imp.shape[1])))
    b1r = b1.reshape(1, D)
    b2r = b2.reshape(1, D)

    return _tc_call(xp, t, a, brow, bcol, wei, wni, weg, Wn_g,
                    W1, b1r, W2, b2r)
